# Initial kernel scaffold; baseline (speedup 1.0000x reference)
import jax
import jax.numpy as jnp
import numpy as np
from jax.experimental import pallas as pl

SH_C0 = 0.28209479177387814
SH_C1 = 0.4886025119029199
SH_C2 = [1.0925484305920792, -1.0925484305920792, 0.31539156525252005,
         -1.0925484305920792, 0.5462742152960396]

NUM_CODES = 4096
CODE_DIM = 32
NUM_NEIGHBORS = 16
SH_BASIS_DIM = 9
NUM_POINTS = 8192
QBLK = 256
IDX_MASK = jnp.int32(~4095)
INT_MAX = jnp.int32(0x7FFFFFFF)


def _tc_body(q_ref, v_ref, cposT_ref, codes_ref, sh_ref, sel_ref,
             out_c_ref, out_s_ref):
    q = q_ref[...]
    acc = None
    for d in range(3):
        diff = q[:, d:d + 1] - cposT_ref[d:d + 1, :]
        acc = diff * diff if acc is None else acc + diff * diff
    iota = jax.lax.broadcasted_iota(jnp.int32, (QBLK, NUM_CODES), 1)
    p = jnp.bitwise_or(
        jnp.bitwise_and(jax.lax.bitcast_convert_type(acc, jnp.int32),
                        IDX_MASK),
        iota)

    W = jnp.zeros((QBLK, NUM_CODES), jnp.float32)
    wsum = jnp.zeros((QBLK, 1), jnp.float32)
    for _ in range(NUM_NEIGHBORS):
        m = jnp.min(p, axis=1, keepdims=True)
        onehot = p == m
        d2m = jax.lax.bitcast_convert_type(jnp.bitwise_and(m, IDX_MASK),
                                           jnp.float32)
        w = 1.0 / (d2m + 1e-16)
        W = W + jnp.where(onehot, w, 0.0)
        wsum = wsum + w
        p = jnp.where(onehot, INT_MAX, p)
    W = W / wsum

    qc = jnp.dot(W, codes_ref[...], preferred_element_type=jnp.float32)
    G = jnp.dot(W, sh_ref[...], preferred_element_type=jnp.float32)

    v = v_ref[...]
    x = v[:, 0:1]
    y = v[:, 1:2]
    z = v[:, 2:3]
    xx, yy, zz = x * x, y * y, z * z
    shb = [
        jnp.full((QBLK, 1), SH_C0, jnp.float32),
        -SH_C1 * y,
        SH_C1 * z,
        -SH_C1 * x,
        SH_C2[0] * (x * y),
        SH_C2[1] * (y * z),
        SH_C2[2] * (2.0 * zz - xx - yy),
        SH_C2[3] * (x * z),
        SH_C2[4] * (xx - yy),
    ]
    bidx = jax.lax.broadcasted_iota(
        jnp.int32, (QBLK, CODE_DIM * SH_BASIS_DIM), 1) % SH_BASIS_DIM
    M = jnp.zeros((QBLK, CODE_DIM * SH_BASIS_DIM), jnp.float32)
    for b in range(SH_BASIS_DIM):
        M = jnp.where(bidx == b, shb[b], M)
    out_s_ref[...] = jnp.dot(G * M, sel_ref[...],
                             preferred_element_type=jnp.float32)
    out_c_ref[...] = qc


def kernel(codes_position, codes, sh_codes, indices, query_points, viewdirs):
    idx0 = indices[0]
    cposT = codes_position[idx0].T
    codes_sel = codes[idx0]
    sh_sel = sh_codes[idx0]
    q = query_points[0]

    sel_np = np.zeros((CODE_DIM * SH_BASIS_DIM, CODE_DIM), np.float32)
    sel_np[np.arange(CODE_DIM * SH_BASIS_DIM),
           np.arange(CODE_DIM * SH_BASIS_DIM) // SH_BASIS_DIM] = 1.0
    sel = jnp.asarray(sel_np)

    grid = (NUM_POINTS // QBLK,)
    out_c, out_s = pl.pallas_call(
        _tc_body,
        grid=grid,
        in_specs=[
            pl.BlockSpec((QBLK, 3), lambda i: (i, 0)),
            pl.BlockSpec((QBLK, 3), lambda i: (i, 0)),
            pl.BlockSpec((3, NUM_CODES), lambda i: (0, 0)),
            pl.BlockSpec((NUM_CODES, CODE_DIM), lambda i: (0, 0)),
            pl.BlockSpec((NUM_CODES, CODE_DIM * SH_BASIS_DIM), lambda i: (0, 0)),
            pl.BlockSpec((CODE_DIM * SH_BASIS_DIM, CODE_DIM), lambda i: (0, 0)),
        ],
        out_specs=[
            pl.BlockSpec((QBLK, CODE_DIM), lambda i: (i, 0)),
            pl.BlockSpec((QBLK, CODE_DIM), lambda i: (i, 0)),
        ],
        out_shape=[
            jax.ShapeDtypeStruct((NUM_POINTS, CODE_DIM), jnp.float32),
            jax.ShapeDtypeStruct((NUM_POINTS, CODE_DIM), jnp.float32),
        ],
    )(q, viewdirs, cposT, codes_sel, sh_sel, sel)
    return (out_c, out_s)


# speedup vs baseline: 5.1507x; 5.1507x over previous
import jax
import jax.numpy as jnp
import numpy as np
from jax.experimental import pallas as pl

SH_C0 = 0.28209479177387814
SH_C1 = 0.4886025119029199
SH_C2 = [1.0925484305920792, -1.0925484305920792, 0.31539156525252005,
         -1.0925484305920792, 0.5462742152960396]

NUM_CODES = 4096
CODE_DIM = 32
NUM_NEIGHBORS = 16
SH_BASIS_DIM = 9
NUM_POINTS = 8192
QBLK = 256
IDX_MASK = ~4095
INT_MAX = 2147483647


def _tc_body(q_ref, v_ref, cposT_ref, codes_ref, sh_ref, sel_ref,
             out_c_ref, out_s_ref):
    q = q_ref[...]
    cposT = cposT_ref[...]
    acc = None
    for d in range(3):
        diff = q[:, d:d + 1] - cposT[d:d + 1, :]
        acc = diff * diff if acc is None else acc + diff * diff
    mm = jnp.dot(q.astype(jnp.bfloat16), cposT.astype(jnp.bfloat16),
                 preferred_element_type=jnp.float32)
    qq = jnp.sum(q * q, axis=1, keepdims=True)
    cc = jnp.sum(cposT * cposT, axis=0, keepdims=True)
    d2sel = qq + cc - 2.0 * mm
    iota = jax.lax.broadcasted_iota(jnp.int32, (QBLK, NUM_CODES), 1)
    p = jnp.bitwise_or(
        jnp.bitwise_and(jax.lax.bitcast_convert_type(d2sel, jnp.int32),
                        IDX_MASK),
        iota)

    def step(_, carry):
        p, W, wsum = carry
        m = jnp.min(p, axis=1, keepdims=True)
        onehot = p == m
        d2e = jnp.min(jnp.where(onehot, acc, jnp.inf), axis=1, keepdims=True)
        w = 1.0 / (d2e + 1e-16)
        W = W + jnp.where(onehot, w, 0.0)
        wsum = wsum + w
        p = jnp.where(onehot, INT_MAX, p)
        return p, W, wsum

    _, W, wsum = jax.lax.fori_loop(
        0, NUM_NEIGHBORS, step,
        (p, jnp.zeros((QBLK, NUM_CODES), jnp.float32),
         jnp.zeros((QBLK, 1), jnp.float32)))
    W = W / wsum

    qc = jnp.dot(W, codes_ref[...], preferred_element_type=jnp.float32)
    G = jnp.dot(W, sh_ref[...], preferred_element_type=jnp.float32)

    v = v_ref[...]
    x = v[:, 0:1]
    y = v[:, 1:2]
    z = v[:, 2:3]
    xx, yy, zz = x * x, y * y, z * z
    shb = [
        jnp.full((QBLK, 1), SH_C0, jnp.float32),
        -SH_C1 * y,
        SH_C1 * z,
        -SH_C1 * x,
        SH_C2[0] * (x * y),
        SH_C2[1] * (y * z),
        SH_C2[2] * (2.0 * zz - xx - yy),
        SH_C2[3] * (x * z),
        SH_C2[4] * (xx - yy),
    ]
    bidx = jax.lax.broadcasted_iota(
        jnp.int32, (QBLK, CODE_DIM * SH_BASIS_DIM), 1) % SH_BASIS_DIM
    M = jnp.zeros((QBLK, CODE_DIM * SH_BASIS_DIM), jnp.float32)
    for b in range(SH_BASIS_DIM):
        M = jnp.where(bidx == b, shb[b], M)
    out_s_ref[...] = jnp.dot(G * M, sel_ref[...],
                             preferred_element_type=jnp.float32)
    out_c_ref[...] = qc


def kernel(codes_position, codes, sh_codes, indices, query_points, viewdirs):
    idx0 = indices[0]
    cposT = codes_position[idx0].T
    codes_sel = codes[idx0]
    sh_sel = sh_codes[idx0]
    q = query_points[0]

    sel_np = np.zeros((CODE_DIM * SH_BASIS_DIM, CODE_DIM), np.float32)
    sel_np[np.arange(CODE_DIM * SH_BASIS_DIM),
           np.arange(CODE_DIM * SH_BASIS_DIM) // SH_BASIS_DIM] = 1.0
    sel = jnp.asarray(sel_np)

    grid = (NUM_POINTS // QBLK,)
    out_c, out_s = pl.pallas_call(
        _tc_body,
        grid=grid,
        in_specs=[
            pl.BlockSpec((QBLK, 3), lambda i: (i, 0)),
            pl.BlockSpec((QBLK, 3), lambda i: (i, 0)),
            pl.BlockSpec((3, NUM_CODES), lambda i: (0, 0)),
            pl.BlockSpec((NUM_CODES, CODE_DIM), lambda i: (0, 0)),
            pl.BlockSpec((NUM_CODES, CODE_DIM * SH_BASIS_DIM), lambda i: (0, 0)),
            pl.BlockSpec((CODE_DIM * SH_BASIS_DIM, CODE_DIM), lambda i: (0, 0)),
        ],
        out_specs=[
            pl.BlockSpec((QBLK, CODE_DIM), lambda i: (i, 0)),
            pl.BlockSpec((QBLK, CODE_DIM), lambda i: (i, 0)),
        ],
        out_shape=[
            jax.ShapeDtypeStruct((NUM_POINTS, CODE_DIM), jnp.float32),
            jax.ShapeDtypeStruct((NUM_POINTS, CODE_DIM), jnp.float32),
        ],
    )(q, viewdirs, cposT, codes_sel, sh_sel, sel)
    return (out_c, out_s)


# speedup vs baseline: 9.9491x; 1.9316x over previous
import jax
import jax.numpy as jnp
import numpy as np
from jax.experimental import pallas as pl

SH_C0 = 0.28209479177387814
SH_C1 = 0.4886025119029199
SH_C2 = [1.0925484305920792, -1.0925484305920792, 0.31539156525252005,
         -1.0925484305920792, 0.5462742152960396]

NUM_CODES = 4096
CODE_DIM = 32
NUM_NEIGHBORS = 16
SH_BASIS_DIM = 9
NUM_POINTS = 8192
QBLK = 256
IDX_MASK = ~4095
INT_MAX = 2147483647


def _tc_body(q_ref, v_ref, cposT_ref, codes_ref, sh_ref, sel_ref,
             out_c_ref, out_s_ref):
    q = q_ref[...]
    cposT = cposT_ref[...]
    acc = None
    for d in range(3):
        diff = q[:, d:d + 1] - cposT[d:d + 1, :]
        acc = diff * diff if acc is None else acc + diff * diff
    mm = jnp.dot(q.astype(jnp.bfloat16), cposT.astype(jnp.bfloat16),
                 preferred_element_type=jnp.float32)
    qq = jnp.sum(q * q, axis=1, keepdims=True)
    cc = jnp.sum(cposT * cposT, axis=0, keepdims=True)
    d2sel = qq + cc - 2.0 * mm
    iota = jax.lax.broadcasted_iota(jnp.int32, (QBLK, NUM_CODES), 1)
    p = jnp.bitwise_or(
        jnp.bitwise_and(jax.lax.bitcast_convert_type(d2sel, jnp.int32),
                        IDX_MASK),
        iota)

    def step(_, p):
        m = jnp.min(p, axis=1, keepdims=True)
        return jnp.where(p == m, INT_MAX, p)

    p = jax.lax.fori_loop(0, NUM_NEIGHBORS, step, p)
    W = jnp.where(p == INT_MAX, 1.0 / (acc + 1e-16), 0.0)
    wsum = jnp.sum(W, axis=1, keepdims=True)

    qc = jnp.dot(W, codes_ref[...], preferred_element_type=jnp.float32) / wsum
    G = jnp.dot(W, sh_ref[...], preferred_element_type=jnp.float32) / wsum

    v = v_ref[...]
    x = v[:, 0:1]
    y = v[:, 1:2]
    z = v[:, 2:3]
    xx, yy, zz = x * x, y * y, z * z
    shb = [
        jnp.full((QBLK, 1), SH_C0, jnp.float32),
        -SH_C1 * y,
        SH_C1 * z,
        -SH_C1 * x,
        SH_C2[0] * (x * y),
        SH_C2[1] * (y * z),
        SH_C2[2] * (2.0 * zz - xx - yy),
        SH_C2[3] * (x * z),
        SH_C2[4] * (xx - yy),
    ]
    bidx = jax.lax.broadcasted_iota(
        jnp.int32, (QBLK, CODE_DIM * SH_BASIS_DIM), 1) % SH_BASIS_DIM
    M = jnp.zeros((QBLK, CODE_DIM * SH_BASIS_DIM), jnp.float32)
    for b in range(SH_BASIS_DIM):
        M = jnp.where(bidx == b, shb[b], M)
    out_s_ref[...] = jnp.dot(G * M, sel_ref[...],
                             preferred_element_type=jnp.float32)
    out_c_ref[...] = qc


def kernel(codes_position, codes, sh_codes, indices, query_points, viewdirs):
    idx0 = indices[0]
    cposT = codes_position[idx0].T
    codes_sel = codes[idx0]
    sh_sel = sh_codes[idx0]
    q = query_points[0]

    sel_np = np.zeros((CODE_DIM * SH_BASIS_DIM, CODE_DIM), np.float32)
    sel_np[np.arange(CODE_DIM * SH_BASIS_DIM),
           np.arange(CODE_DIM * SH_BASIS_DIM) // SH_BASIS_DIM] = 1.0
    sel = jnp.asarray(sel_np)

    grid = (NUM_POINTS // QBLK,)
    out_c, out_s = pl.pallas_call(
        _tc_body,
        grid=grid,
        in_specs=[
            pl.BlockSpec((QBLK, 3), lambda i: (i, 0)),
            pl.BlockSpec((QBLK, 3), lambda i: (i, 0)),
            pl.BlockSpec((3, NUM_CODES), lambda i: (0, 0)),
            pl.BlockSpec((NUM_CODES, CODE_DIM), lambda i: (0, 0)),
            pl.BlockSpec((NUM_CODES, CODE_DIM * SH_BASIS_DIM), lambda i: (0, 0)),
            pl.BlockSpec((CODE_DIM * SH_BASIS_DIM, CODE_DIM), lambda i: (0, 0)),
        ],
        out_specs=[
            pl.BlockSpec((QBLK, CODE_DIM), lambda i: (i, 0)),
            pl.BlockSpec((QBLK, CODE_DIM), lambda i: (i, 0)),
        ],
        out_shape=[
            jax.ShapeDtypeStruct((NUM_POINTS, CODE_DIM), jnp.float32),
            jax.ShapeDtypeStruct((NUM_POINTS, CODE_DIM), jnp.float32),
        ],
    )(q, viewdirs, cposT, codes_sel, sh_sel, sel)
    return (out_c, out_s)
